# baseline (speedup 1.0000x reference)
import jax
import jax.numpy as jnp
from jax.experimental import pallas as pl

_N = 8192
_L = 128
_C = _N // _L
_BLK = 1024


def _vq_kernel(x_ref, w_ref, val_ref, idx_ref):
    xv = x_ref[...]
    lanes = jax.lax.broadcasted_iota(jnp.int32, (_BLK, _L), 1)

    def body(j, carry):
        bd, bj, bv = carry
        codes = w_ref[j, :].reshape(1, _L)
        d = xv - codes
        d = d * d
        m = d < bd
        bd = jnp.where(m, d, bd)
        bj = jnp.where(m, j, bj)
        bv = jnp.where(m, jnp.broadcast_to(codes, (_BLK, _L)), bv)
        return bd, bj, bv

    bd0 = jnp.full((_BLK, _L), jnp.inf, jnp.float32)
    bj0 = jnp.zeros((_BLK, _L), jnp.int32)
    bv0 = jnp.zeros((_BLK, _L), jnp.float32)
    bd, bj, bv = jax.lax.fori_loop(0, _C, body, (bd0, bj0, bv0))

    bi = bj * _L + lanes
    md = jnp.min(bd, axis=1, keepdims=True)
    big = jnp.int32(2**30)
    mi = jnp.min(jnp.where(bd == md, bi, big), axis=1, keepdims=True)
    mv = jnp.max(jnp.where(bi == mi, bv, -jnp.inf), axis=1, keepdims=True)
    idx_ref[...] = mi
    val_ref[...] = mv


def kernel(x, weight):
    shape = x.shape
    xf = x.reshape(_N, 1)
    wf = weight.reshape(_C, _L)
    val, idx = pl.pallas_call(
        _vq_kernel,
        grid=(_N // _BLK,),
        in_specs=[
            pl.BlockSpec((_BLK, 1), lambda i: (i, 0)),
            pl.BlockSpec((_C, _L), lambda i: (0, 0)),
        ],
        out_specs=[
            pl.BlockSpec((_BLK, 1), lambda i: (i, 0)),
            pl.BlockSpec((_BLK, 1), lambda i: (i, 0)),
        ],
        out_shape=[
            jax.ShapeDtypeStruct((_N, 1), jnp.float32),
            jax.ShapeDtypeStruct((_N, 1), jnp.int32),
        ],
    )(xf, wf)
    return val.reshape(shape), idx.reshape(shape)


# speedup vs baseline: 1.0034x; 1.0034x over previous
import jax
import jax.numpy as jnp
from jax.experimental import pallas as pl
from jax.experimental.pallas import tpu as pltpu

_N = 8192
_L = 128
_C = _N // _L
_BLK = 1024


def _vq_kernel(x_ref, w_ref, val_ref, idx_ref):
    xv = x_ref[...]
    lanes = jax.lax.broadcasted_iota(jnp.int32, (_BLK, _L), 1)

    def body(j, carry):
        bd, bj, bv = carry
        codes = w_ref[j, :].reshape(1, _L)
        d = xv - codes
        d = d * d
        m = d < bd
        bd = jnp.where(m, d, bd)
        bj = jnp.where(m, j, bj)
        bv = jnp.where(m, jnp.broadcast_to(codes, (_BLK, _L)), bv)
        return bd, bj, bv

    bd0 = jnp.full((_BLK, _L), jnp.inf, jnp.float32)
    bj0 = jnp.zeros((_BLK, _L), jnp.int32)
    bv0 = jnp.zeros((_BLK, _L), jnp.float32)
    bd, bj, bv = jax.lax.fori_loop(0, _C, body, (bd0, bj0, bv0))

    bi = bj * _L + lanes
    md = jnp.min(bd, axis=1, keepdims=True)
    big = jnp.int32(2**30)
    mi = jnp.min(jnp.where(bd == md, bi, big), axis=1, keepdims=True)
    mv = jnp.max(jnp.where(bi == mi, bv, -jnp.inf), axis=1, keepdims=True)
    idx_ref[...] = mi
    val_ref[...] = mv


def kernel(x, weight):
    shape = x.shape
    xf = x.reshape(_N, 1)
    wf = weight.reshape(_C, _L)
    val, idx = pl.pallas_call(
        _vq_kernel,
        grid=(_N // _BLK,),
        in_specs=[
            pl.BlockSpec((_BLK, 1), lambda i: (i, 0)),
            pl.BlockSpec((_C, _L), lambda i: (0, 0)),
        ],
        out_specs=[
            pl.BlockSpec((_BLK, 1), lambda i: (i, 0)),
            pl.BlockSpec((_BLK, 1), lambda i: (i, 0)),
        ],
        out_shape=[
            jax.ShapeDtypeStruct((_N, 1), jnp.float32),
            jax.ShapeDtypeStruct((_N, 1), jnp.int32),
        ],
        compiler_params=pltpu.CompilerParams(
            dimension_semantics=("parallel",),
        ),
    )(xf, wf)
    return val.reshape(shape), idx.reshape(shape)


# speedup vs baseline: 3.3476x; 3.3362x over previous
import jax
import jax.numpy as jnp
from jax.experimental import pallas as pl
from jax.experimental.pallas import tpu as pltpu

_N = 8192
_R = 64
_L = 128
_U = 8


def _vq_kernel(w_ref, x_ref, val_ref, idx_ref):
    xv = x_ref[...]

    def body(t, carry):
        bd, bj, bv = carry
        for u in range(_U):
            j = t * _U + u
            c = w_ref[j]
            d = xv - c
            d = d * d
            m = d < bd
            bd = jnp.where(m, d, bd)
            bj = jnp.where(m, j, bj)
            bv = jnp.where(m, c, bv)
        return bd, bj, bv

    bd0 = jnp.full((_R, _L), jnp.inf, jnp.float32)
    bj0 = jnp.zeros((_R, _L), jnp.int32)
    bv0 = jnp.zeros((_R, _L), jnp.float32)
    _, bj, bv = jax.lax.fori_loop(0, _N // _U, body, (bd0, bj0, bv0))

    idx_ref[...] = bj
    val_ref[...] = bv


def kernel(x, weight):
    shape = x.shape
    xf = x.reshape(_R, _L)
    wf = weight.reshape(_N)
    val, idx = pl.pallas_call(
        _vq_kernel,
        in_specs=[
            pl.BlockSpec(memory_space=pltpu.MemorySpace.SMEM),
            pl.BlockSpec(memory_space=pltpu.MemorySpace.VMEM),
        ],
        out_specs=[
            pl.BlockSpec(memory_space=pltpu.MemorySpace.VMEM),
            pl.BlockSpec(memory_space=pltpu.MemorySpace.VMEM),
        ],
        out_shape=[
            jax.ShapeDtypeStruct((_R, _L), jnp.float32),
            jax.ShapeDtypeStruct((_R, _L), jnp.int32),
        ],
    )(wf, xf)
    return val.reshape(shape), idx.reshape(shape)


# speedup vs baseline: 3.7212x; 1.1116x over previous
import jax
import jax.numpy as jnp
from jax.experimental import pallas as pl
from jax.experimental.pallas import tpu as pltpu

_N = 8192
_R = 64
_L = 128
_U = 64


def _vq_kernel(w_ref, x_ref, val_ref, idx_ref):
    xv = x_ref[...]

    def body(t, carry):
        bd, bj, bv = carry
        for u in range(_U):
            j = t * _U + u
            c = w_ref[j]
            d = xv - c
            d = d * d
            m = d < bd
            bd = jnp.where(m, d, bd)
            bj = jnp.where(m, j, bj)
            bv = jnp.where(m, c, bv)
        return bd, bj, bv

    bd0 = jnp.full((_R, _L), jnp.inf, jnp.float32)
    bj0 = jnp.zeros((_R, _L), jnp.int32)
    bv0 = jnp.zeros((_R, _L), jnp.float32)
    _, bj, bv = jax.lax.fori_loop(0, _N // _U, body, (bd0, bj0, bv0))

    idx_ref[...] = bj
    val_ref[...] = bv


def kernel(x, weight):
    shape = x.shape
    xf = x.reshape(_R, _L)
    wf = weight.reshape(_N)
    val, idx = pl.pallas_call(
        _vq_kernel,
        in_specs=[
            pl.BlockSpec(memory_space=pltpu.MemorySpace.SMEM),
            pl.BlockSpec(memory_space=pltpu.MemorySpace.VMEM),
        ],
        out_specs=[
            pl.BlockSpec(memory_space=pltpu.MemorySpace.VMEM),
            pl.BlockSpec(memory_space=pltpu.MemorySpace.VMEM),
        ],
        out_shape=[
            jax.ShapeDtypeStruct((_R, _L), jnp.float32),
            jax.ShapeDtypeStruct((_R, _L), jnp.int32),
        ],
    )(wf, xf)
    return val.reshape(shape), idx.reshape(shape)
